# baseline (speedup 1.0000x reference)
import functools
import math

import jax
import jax.numpy as jnp
from jax import lax
from jax.experimental import pallas as pl
from jax.experimental.pallas import tpu as pltpu
from jax.experimental.pallas import tpu_sc as plsc

B_ = 16384
A_ = 26
NB_ = 20
D_ = 32
OUT_ = 128
H_ = (A_ * D_) // 2
C_ = A_ * NB_
AD_ = A_ * D_

BT = 4096
NT = B_ // BT

L_ = 16
NW_ = 32
ROWS_ = B_ // NW_
OFF_ = A_ - L_

_INV_SQRT2 = 1.0 / math.sqrt(2.0)


def _gelu(x):
    return 0.5 * x * (1.0 + jax.lax.erf(x * _INV_SQRT2))


def _sc_minmax_body(act_hbm, out_hbm, act_v, part_v):
    c = lax.axis_index("c")
    s = lax.axis_index("s")
    wid = s * 2 + c

    pltpu.sync_copy(act_hbm.at[pl.ds(wid * (ROWS_ * A_), ROWS_ * A_)], act_v)

    def p1(i, carry):
        mn0, mn1, mx0, mx1 = carry
        a0 = act_v[pl.ds(i * A_, L_)]
        a1 = act_v[pl.ds(i * A_ + OFF_, L_)]
        return (jnp.minimum(mn0, a0), jnp.minimum(mn1, a1),
                jnp.maximum(mx0, a0), jnp.maximum(mx1, a1))

    big = jnp.full((L_,), jnp.inf, jnp.float32)
    mn0, mn1, mx0, mx1 = lax.fori_loop(0, ROWS_, p1, (big, big, -big, -big),
                                       unroll=8)
    part_v[pl.ds(0, L_)] = mn0
    part_v[pl.ds(L_, L_)] = mn1
    part_v[pl.ds(2 * L_, L_)] = mx0
    part_v[pl.ds(3 * L_, L_)] = mx1
    pltpu.sync_copy(part_v.at[pl.ds(0, 2 * L_)],
                    out_hbm.at[pl.ds(wid * (2 * L_), 2 * L_)])
    pltpu.sync_copy(part_v.at[pl.ds(2 * L_, 2 * L_)],
                    out_hbm.at[pl.ds((NW_ + wid) * (2 * L_), 2 * L_)])


@functools.partial(
    pl.kernel,
    mesh=plsc.VectorSubcoreMesh(core_axis_name="c", subcore_axis_name="s"),
    out_type=jax.ShapeDtypeStruct((2 * NW_ * 2 * L_,), jnp.float32),
    scratch_types=[
        pltpu.VMEM((ROWS_ * A_,), jnp.float32),
        pltpu.VMEM((4 * L_,), jnp.float32),
    ],
)
def _sc_minmax(act_hbm, out_hbm, act_v, part_v):
    _sc_minmax_body(act_hbm, out_hbm, act_v, part_v)


def _main_body(tlin_ref, pm_ref, act_ref, tab_ref, W1_ref, b1_ref, W2_ref,
               b2_ref, out_ref, mm_ref, E_ref, M_ref):
    t = pl.program_id(0)

    @pl.when(t == 0)
    def _prep():
        mna = jnp.min(pm_ref[0:NW_, :], axis=0, keepdims=True)
        mxa = jnp.max(pm_ref[NW_:2 * NW_, :], axis=0, keepdims=True)
        mn26 = jnp.concatenate(
            [mna[0:1, 0:L_], mna[0:1, L_ + (L_ - OFF_):2 * L_]], axis=1)
        mx26 = jnp.concatenate(
            [mxa[0:1, 0:L_], mxa[0:1, L_ + (L_ - OFF_):2 * L_]], axis=1)
        mm_ref[0:1, :] = mn26
        mm_ref[1:2, :] = mx26 - mn26
        er = jax.lax.broadcasted_iota(jnp.int32, (A_, C_), 0)
        ec = jax.lax.broadcasted_iota(jnp.int32, (A_, C_), 1)
        E_ref[...] = jnp.where(ec // NB_ == er, 1.0, 0.0).astype(jnp.bfloat16)
        dr = jax.lax.broadcasted_iota(jnp.int32, (D_, AD_), 0)
        dc = jax.lax.broadcasted_iota(jnp.int32, (D_, AD_), 1)
        erep = jnp.where(dc % D_ == dr, 1.0, 0.0)
        mr = jax.lax.broadcasted_iota(jnp.int32, (C_, AD_), 0)
        mc = jax.lax.broadcasted_iota(jnp.int32, (C_, AD_), 1)
        mask = jnp.where(mr // NB_ == mc // D_, 1.0, 0.0)
        t520 = jnp.dot(tab_ref[...], erep,
                       preferred_element_type=jnp.float32) * mask
        M_ref[...] = jnp.dot(t520, W1_ref[...],
                             preferred_element_type=jnp.float32
                             ).astype(jnp.bfloat16)

    act = act_ref[...]
    mn = mm_ref[0:1, :]
    diff = mm_ref[1:2, :]
    cnt = jnp.zeros_like(act)
    for k in range(1, NB_ + 1):
        th = mn + diff * tlin_ref[0, k]
        cnt = cnt + jnp.where(th < act, 1.0, 0.0)
    binv = jnp.minimum(cnt, float(NB_ - 1)).astype(jnp.bfloat16)
    bin_e = jnp.dot(binv, E_ref[...], preferred_element_type=jnp.float32)
    cidx = jax.lax.broadcasted_iota(jnp.int32, (1, C_), 1)
    jmod = (cidx % NB_).astype(jnp.float32)
    onehot = jnp.where(bin_e == jmod, 1.0, 0.0).astype(jnp.bfloat16)
    hpre = jnp.dot(onehot, M_ref[...],
                   preferred_element_type=jnp.float32) + b1_ref[...]
    h = _gelu(hpre).astype(jnp.bfloat16)
    o = jnp.dot(h, W2_ref[...], preferred_element_type=jnp.float32)
    out_ref[...] = _gelu(o + b2_ref[...])


def kernel(actions, tables, W1, b1, W2, b2):
    tab520 = tables.reshape(C_, D_)
    tlin2 = jnp.linspace(0.0, 1.0, NB_ + 1,
                         dtype=jnp.float32).reshape(1, NB_ + 1)
    b1r = b1.reshape(1, H_)
    b2r = b2.reshape(1, OUT_)

    pm = _sc_minmax(actions.reshape(-1)).reshape(2 * NW_, 2 * L_)

    out = pl.pallas_call(
        _main_body,
        grid=(NT,),
        in_specs=[
            pl.BlockSpec((1, NB_ + 1), lambda t: (0, 0)),
            pl.BlockSpec((2 * NW_, 2 * L_), lambda t: (0, 0)),
            pl.BlockSpec((BT, A_), lambda t: (t, 0)),
            pl.BlockSpec((C_, D_), lambda t: (0, 0)),
            pl.BlockSpec((AD_, H_), lambda t: (0, 0)),
            pl.BlockSpec((1, H_), lambda t: (0, 0)),
            pl.BlockSpec((H_, OUT_), lambda t: (0, 0)),
            pl.BlockSpec((1, OUT_), lambda t: (0, 0)),
        ],
        out_specs=pl.BlockSpec((BT, OUT_), lambda t: (t, 0)),
        out_shape=jax.ShapeDtypeStruct((B_, OUT_), jnp.float32),
        scratch_shapes=[
            pltpu.VMEM((2, A_), jnp.float32),
            pltpu.VMEM((A_, C_), jnp.bfloat16),
            pltpu.VMEM((C_, H_), jnp.bfloat16),
        ],
        compiler_params=pltpu.CompilerParams(
            dimension_semantics=("arbitrary",)),
    )(tlin2, pm, actions, tab520, W1, b1r, W2, b2r)
    return out


# speedup vs baseline: 1.3661x; 1.3661x over previous
import math

import jax
import jax.numpy as jnp
from jax.experimental import pallas as pl
from jax.experimental.pallas import tpu as pltpu

B_ = 16384
A_ = 26
NB_ = 20
D_ = 32
OUT_ = 128
H_ = (A_ * D_) // 2
C_ = A_ * NB_
AD_ = A_ * D_

BT = 4096
NT = B_ // BT

_INV_SQRT2 = 1.0 / math.sqrt(2.0)


def _gelu(x):
    return 0.5 * x * (1.0 + jax.lax.erf(x * _INV_SQRT2))


def _minmax_body(act_ref, mm_ref):
    t = pl.program_id(0)
    act = act_ref[...]
    mn = jnp.min(act, axis=0, keepdims=True)
    mx = jnp.max(act, axis=0, keepdims=True)
    cur = jnp.concatenate([mn, -mx], axis=0)

    @pl.when(t == 0)
    def _init():
        mm_ref[...] = cur

    @pl.when(t != 0)
    def _acc():
        mm_ref[...] = jnp.minimum(mm_ref[...], cur)


def _main_body(tlin_ref, mm_ref, act_ref, tab_ref, W1_ref, b1_ref, W2_ref,
               b2_ref, out_ref, E_ref, M_ref):
    t = pl.program_id(0)

    @pl.when(t == 0)
    def _prep():
        er = jax.lax.broadcasted_iota(jnp.int32, (A_, C_), 0)
        ec = jax.lax.broadcasted_iota(jnp.int32, (A_, C_), 1)
        E_ref[...] = jnp.where(ec // NB_ == er, 1.0, 0.0).astype(jnp.bfloat16)
        dr = jax.lax.broadcasted_iota(jnp.int32, (D_, AD_), 0)
        dc = jax.lax.broadcasted_iota(jnp.int32, (D_, AD_), 1)
        erep = jnp.where(dc % D_ == dr, 1.0, 0.0)
        mr = jax.lax.broadcasted_iota(jnp.int32, (C_, AD_), 0)
        mc = jax.lax.broadcasted_iota(jnp.int32, (C_, AD_), 1)
        mask = jnp.where(mr // NB_ == mc // D_, 1.0, 0.0)
        t520 = jnp.dot(tab_ref[...], erep,
                       preferred_element_type=jnp.float32) * mask
        M_ref[...] = jnp.dot(t520, W1_ref[...],
                             preferred_element_type=jnp.float32
                             ).astype(jnp.bfloat16)

    act = act_ref[...]
    mn = mm_ref[0:1, :]
    diff = (-mm_ref[1:2, :]) - mn
    cnt = jnp.zeros_like(act)
    for k in range(1, NB_ + 1):
        th = mn + diff * tlin_ref[0, k]
        cnt = cnt + jnp.where(th < act, 1.0, 0.0)
    binv = jnp.minimum(cnt, float(NB_ - 1)).astype(jnp.bfloat16)
    bin_e = jnp.dot(binv, E_ref[...], preferred_element_type=jnp.float32)
    cidx = jax.lax.broadcasted_iota(jnp.int32, (1, C_), 1)
    jmod = (cidx % NB_).astype(jnp.float32)
    onehot = jnp.where(bin_e == jmod, 1.0, 0.0).astype(jnp.bfloat16)
    hpre = jnp.dot(onehot, M_ref[...],
                   preferred_element_type=jnp.float32) + b1_ref[...]
    h = _gelu(hpre).astype(jnp.bfloat16)
    o = jnp.dot(h, W2_ref[...], preferred_element_type=jnp.float32)
    out_ref[...] = _gelu(o + b2_ref[...])


def kernel(actions, tables, W1, b1, W2, b2):
    tab520 = tables.reshape(C_, D_)
    tlin2 = jnp.linspace(0.0, 1.0, NB_ + 1,
                         dtype=jnp.float32).reshape(1, NB_ + 1)
    b1r = b1.reshape(1, H_)
    b2r = b2.reshape(1, OUT_)

    mm = pl.pallas_call(
        _minmax_body,
        grid=(NT,),
        in_specs=[pl.BlockSpec((BT, A_), lambda t: (t, 0))],
        out_specs=pl.BlockSpec((2, A_), lambda t: (0, 0)),
        out_shape=jax.ShapeDtypeStruct((2, A_), jnp.float32),
        compiler_params=pltpu.CompilerParams(
            dimension_semantics=("arbitrary",)),
    )(actions)

    out = pl.pallas_call(
        _main_body,
        grid=(NT,),
        in_specs=[
            pl.BlockSpec((1, NB_ + 1), lambda t: (0, 0)),
            pl.BlockSpec((2, A_), lambda t: (0, 0)),
            pl.BlockSpec((BT, A_), lambda t: (t, 0)),
            pl.BlockSpec((C_, D_), lambda t: (0, 0)),
            pl.BlockSpec((AD_, H_), lambda t: (0, 0)),
            pl.BlockSpec((1, H_), lambda t: (0, 0)),
            pl.BlockSpec((H_, OUT_), lambda t: (0, 0)),
            pl.BlockSpec((1, OUT_), lambda t: (0, 0)),
        ],
        out_specs=pl.BlockSpec((BT, OUT_), lambda t: (t, 0)),
        out_shape=jax.ShapeDtypeStruct((B_, OUT_), jnp.float32),
        scratch_shapes=[
            pltpu.VMEM((A_, C_), jnp.bfloat16),
            pltpu.VMEM((C_, H_), jnp.bfloat16),
        ],
        compiler_params=pltpu.CompilerParams(
            dimension_semantics=("arbitrary",)),
    )(tlin2, mm, actions, tab520, W1, b1r, W2, b2r)
    return out
